# baseline (speedup 1.0000x reference)
import functools

import jax
import jax.numpy as jnp
from jax import lax
from jax.experimental import pallas as pl
from jax.experimental.pallas import tpu as pltpu
from jax.experimental.pallas import tpu_sc as plsc

_N, _C, _H, _W = 4, 128, 224, 224
_IMGS = _N * _C
_NTEC = 32
_PER = _IMGS // _NTEC
_HH = _H // 2

_mesh = plsc.VectorSubcoreMesh(core_axis_name="c", subcore_axis_name="s")


@functools.partial(
    pl.kernel,
    out_type=jax.ShapeDtypeStruct((_N, _C, _H, _W), jnp.float32),
    mesh=_mesh,
    scratch_types=[
        pltpu.VMEM((_HH, _W), jnp.float32),
        pltpu.VMEM((_HH, _W), jnp.float32),
        pltpu.VMEM((_HH, _W), jnp.float32),
        pltpu.VMEM((_HH, _W), jnp.float32),
        pltpu.SemaphoreType.DMA,
        pltpu.SemaphoreType.DMA,
        pltpu.SemaphoreType.DMA,
        pltpu.SemaphoreType.DMA,
    ],
)
def _block_relu(x_hbm, y_hbm, in0, in1, out0, out1, si0, si1, so0, so1):
    g = lax.axis_index("s") * 2 + lax.axis_index("c")
    typ = g % 8

    lane = lax.iota(jnp.int32, 16)
    e0 = lane & -2
    e1 = e0 | 1
    q0 = lane & -4
    q1 = q0 | 1
    q2 = q0 | 2
    q3 = q0 | 3

    def gat(v, idx):
        return v.at[idx].get(mode="promise_in_bounds")

    def src(p, b):
        img = g * _PER + p
        n = img // _C
        c = img % _C
        return x_hbm.at[n, c, pl.ds(b * _HH, _HH)]

    def dst(p, b):
        img = g * _PER + p
        n = img // _C
        c = img % _C
        return y_hbm.at[n, c, pl.ds(b * _HH, _HH)]

    def compute_2x2(ib, ob):
        def rowp(hp, carry):
            r0 = hp * 2
            r1 = r0 + 1
            for j in range(_W // 16):
                cs = pl.ds(j * 16, 16)
                a = ib[r0, cs]
                b = ib[r1, cs]
                t = a + b
                s = gat(t, e0) + gat(t, e1)
                keep = s >= 0.0
                ob[r0, cs] = jnp.where(keep, a, 0.0)
                ob[r1, cs] = jnp.where(keep, b, 0.0)
            return carry
        lax.fori_loop(0, _HH // 2, rowp, 0)

    def compute_4x4(ib, ob):
        def rowq(hq, carry):
            r0 = hq * 4
            for j in range(_W // 16):
                cs = pl.ds(j * 16, 16)
                v0 = ib[r0, cs]
                v1 = ib[r0 + 1, cs]
                v2 = ib[r0 + 2, cs]
                v3 = ib[r0 + 3, cs]
                t = (v0 + v1) + (v2 + v3)
                s = (gat(t, q0) + gat(t, q1)) + (gat(t, q2) + gat(t, q3))
                keep = s >= 0.0
                ob[r0, cs] = jnp.where(keep, v0, 0.0)
                ob[r0 + 1, cs] = jnp.where(keep, v1, 0.0)
                ob[r0 + 2, cs] = jnp.where(keep, v2, 0.0)
                ob[r0 + 3, cs] = jnp.where(keep, v3, 0.0)
            return carry
        lax.fori_loop(0, _HH // 4, rowq, 0)

    def identity_path():
        def img_body(p, carry):
            img = g * _PER + p
            n = img // _C
            c = img % _C
            pltpu.sync_copy(x_hbm.at[n, c], y_hbm.at[n, c])
            return carry
        lax.fori_loop(0, _PER, img_body, 0)

    def compute_path():
        ins = (in0, in1)
        outs = (out0, out1)
        sins = (si0, si1)
        souts = (so0, so1)

        for b in range(2):
            pltpu.make_async_copy(src(0, b), ins[b], sins[b]).start()

        def step(p, carry):
            for b in range(2):
                ib, ob, si, so = ins[b], outs[b], sins[b], souts[b]
                pltpu.make_async_copy(src(p, b), ib, si).wait()
                pl.when(p > 0)(
                    lambda ob=ob, so=so, p=p, b=b:
                        pltpu.make_async_copy(ob, dst(p - 1, b), so).wait())
                pl.when(typ < 4)(lambda ib=ib, ob=ob: compute_2x2(ib, ob))
                pl.when(typ >= 4)(lambda ib=ib, ob=ob: compute_4x4(ib, ob))
                pltpu.make_async_copy(ob, dst(p, b), so).start()
                pl.when(p < _PER - 1)(
                    lambda ib=ib, si=si, p=p, b=b:
                        pltpu.make_async_copy(src(p + 1, b), ib, si).start())
            return carry

        lax.fori_loop(0, _PER, step, 0)

        for b in range(2):
            pltpu.make_async_copy(outs[b], dst(_PER - 1, b), souts[b]).wait()

    pl.when(typ == 7)(identity_path)
    pl.when(typ < 7)(compute_path)


def kernel(activation):
    return _block_relu(activation)


# speedup vs baseline: 1.9508x; 1.9508x over previous
import functools

import jax
import jax.numpy as jnp
from jax import lax
from jax.experimental import pallas as pl
from jax.experimental.pallas import tpu as pltpu
from jax.experimental.pallas import tpu_sc as plsc

_N, _C, _H, _W = 4, 128, 224, 224
_IMGS = _N * _C
_NTEC = 32
_PER = _IMGS // _NTEC
_HH = _H // 2

_mesh = plsc.VectorSubcoreMesh(core_axis_name="c", subcore_axis_name="s")


@functools.partial(
    pl.kernel,
    out_type=jax.ShapeDtypeStruct((_N, _C, _H, _W), jnp.float32),
    mesh=_mesh,
    scratch_types=[
        pltpu.VMEM((_HH, _W), jnp.float32),
        pltpu.VMEM((_HH, _W), jnp.float32),
        pltpu.VMEM((_HH, _W), jnp.float32),
        pltpu.VMEM((_HH, _W), jnp.float32),
        pltpu.SemaphoreType.DMA,
        pltpu.SemaphoreType.DMA,
        pltpu.SemaphoreType.DMA,
        pltpu.SemaphoreType.DMA,
    ],
)
def _block_relu(x_hbm, y_hbm, in0, in1, out0, out1, si0, si1, so0, so1):
    g = lax.axis_index("s") * 2 + lax.axis_index("c")
    typ = g % 8

    lane = lax.iota(jnp.int32, 16)
    e0 = lane & -2
    e1 = e0 | 1
    q0 = lane & -4
    q1 = q0 | 1
    q2 = q0 | 2
    q3 = q0 | 3

    def gat(v, idx):
        return v.at[idx].get(mode="promise_in_bounds")

    def src(p, b):
        img = g * _PER + p
        n = img // _C
        c = img % _C
        return x_hbm.at[n, c, pl.ds(b * _HH, _HH)]

    def dst(p, b):
        img = g * _PER + p
        n = img // _C
        c = img % _C
        return y_hbm.at[n, c, pl.ds(b * _HH, _HH)]

    def compute_2x2(ib, ob):
        def rowp(hp, carry):
            r0 = hp * 2
            r1 = r0 + 1
            for j in range(_W // 16):
                cs = pl.ds(j * 16, 16)
                a = ib[r0, cs]
                b = ib[r1, cs]
                t = a + b
                s = gat(t, e0) + gat(t, e1)
                keep = s >= 0.0
                ob[r0, cs] = jnp.where(keep, a, 0.0)
                ob[r1, cs] = jnp.where(keep, b, 0.0)
            return carry
        lax.fori_loop(0, _HH // 2, rowp, 0)

    def compute_4x4(ib, ob):
        def rowq(hq, carry):
            r0 = hq * 4
            for j in range(_W // 16):
                cs = pl.ds(j * 16, 16)
                v0 = ib[r0, cs]
                v1 = ib[r0 + 1, cs]
                v2 = ib[r0 + 2, cs]
                v3 = ib[r0 + 3, cs]
                t = (v0 + v1) + (v2 + v3)
                s = (gat(t, q0) + gat(t, q1)) + (gat(t, q2) + gat(t, q3))
                keep = s >= 0.0
                ob[r0, cs] = jnp.where(keep, v0, 0.0)
                ob[r0 + 1, cs] = jnp.where(keep, v1, 0.0)
                ob[r0 + 2, cs] = jnp.where(keep, v2, 0.0)
                ob[r0 + 3, cs] = jnp.where(keep, v3, 0.0)
            return carry
        lax.fori_loop(0, _HH // 4, rowq, 0)

    def copy_rows(ib, ob):
        def rowc(r, carry):
            for j in range(_W // 16):
                cs = pl.ds(j * 16, 16)
                ob[r, cs] = ib[r, cs]
            return carry
        lax.fori_loop(0, _HH, rowc, 0)

    def compute_path():
        ins = (in0, in1)
        outs = (out0, out1)
        sins = (si0, si1)
        souts = (so0, so1)

        for b in range(2):
            pltpu.make_async_copy(src(0, b), ins[b], sins[b]).start()

        def step(p, carry):
            for b in range(2):
                ib, ob, si, so = ins[b], outs[b], sins[b], souts[b]
                pltpu.make_async_copy(src(p, b), ib, si).wait()
                pl.when(p > 0)(
                    lambda ob=ob, so=so, p=p, b=b:
                        pltpu.make_async_copy(ob, dst(p - 1, b), so).wait())
                pl.when(typ < 4)(lambda ib=ib, ob=ob: compute_2x2(ib, ob))
                pl.when((typ >= 4) & (typ < 7))(
                    lambda ib=ib, ob=ob: compute_4x4(ib, ob))
                pl.when(typ == 7)(lambda ib=ib, ob=ob: copy_rows(ib, ob))
                pltpu.make_async_copy(ob, dst(p, b), so).start()
                pl.when(p < _PER - 1)(
                    lambda ib=ib, si=si, p=p, b=b:
                        pltpu.make_async_copy(src(p + 1, b), ib, si).start())
            return carry

        lax.fori_loop(0, _PER, step, 0)

        for b in range(2):
            pltpu.make_async_copy(outs[b], dst(_PER - 1, b), souts[b]).wait()

    compute_path()


def kernel(activation):
    return _block_relu(activation)


# speedup vs baseline: 7.3552x; 3.7703x over previous
import functools

import jax
import jax.numpy as jnp
from jax import lax
from jax.experimental import pallas as pl
from jax.experimental.pallas import tpu as pltpu
from jax.experimental.pallas import tpu_sc as plsc

_N, _C, _H, _W = 4, 128, 224, 224
_NTEC = 32
_SLABS = _NTEC // _N
_SLAB_H = _H // _SLABS
_CH, _CW = 4, 56
_NQ = _SLAB_H // _CH
_NW = _W // _CW
_CHUNKS = _NQ * _NW
_PAIRS = _CHUNKS // 2

_mesh = plsc.VectorSubcoreMesh(core_axis_name="c", subcore_axis_name="s")


@functools.partial(
    pl.kernel,
    out_type=jax.ShapeDtypeStruct((_N, _H, _W, _C), jnp.float32),
    mesh=_mesh,
    scratch_types=[
        pltpu.VMEM((_CH, _CW, _C), jnp.float32),
        pltpu.VMEM((_CH, _CW, _C), jnp.float32),
        pltpu.VMEM((_CH, _CW, _C), jnp.float32),
        pltpu.VMEM((_CH, _CW, _C), jnp.float32),
        pltpu.SemaphoreType.DMA,
        pltpu.SemaphoreType.DMA,
        pltpu.SemaphoreType.DMA,
        pltpu.SemaphoreType.DMA,
    ],
)
def _block_relu_nhwc(x_hbm, y_hbm, in0, in1, out0, out1, si0, si1, so0, so1):
    g = lax.axis_index("s") * 2 + lax.axis_index("c")
    n = g // _SLABS
    row0 = (g % _SLABS) * _SLAB_H

    def chunk_slice(ref, k):
        q = k // _NW
        w4 = k % _NW
        return ref.at[n, pl.ds(row0 + q * _CH, _CH), pl.ds(w4 * _CW, _CW)]

    def compute(ib, ob):
        def w2_body(w2, carry):
            c0 = w2 * 2
            c1 = c0 + 1
            for r0 in (0, 2):
                r1 = r0 + 1
                for j in range(4):
                    cs = pl.ds(j * 16, 16)
                    a = ib[r0, c0, cs]
                    b = ib[r0, c1, cs]
                    c = ib[r1, c0, cs]
                    d = ib[r1, c1, cs]
                    s = (a + b) + (c + d)
                    keep = s >= 0.0
                    ob[r0, c0, cs] = jnp.where(keep, a, 0.0)
                    ob[r0, c1, cs] = jnp.where(keep, b, 0.0)
                    ob[r1, c0, cs] = jnp.where(keep, c, 0.0)
                    ob[r1, c1, cs] = jnp.where(keep, d, 0.0)
            return carry
        lax.fori_loop(0, _CW // 2, w2_body, 0)

        def w4_body(w4, carry):
            cb = w4 * 4
            for j in range(3):
                cs = pl.ds(64 + j * 16, 16)
                v = [ib[r, cb + c, cs] for r in range(4) for c in range(4)]
                s01 = (v[0] + v[1]) + (v[2] + v[3])
                s23 = (v[4] + v[5]) + (v[6] + v[7])
                s45 = (v[8] + v[9]) + (v[10] + v[11])
                s67 = (v[12] + v[13]) + (v[14] + v[15])
                s = (s01 + s23) + (s45 + s67)
                keep = s >= 0.0
                for r in range(4):
                    for c in range(4):
                        ob[r, cb + c, cs] = jnp.where(keep, v[r * 4 + c], 0.0)
            return carry
        lax.fori_loop(0, _CW // 4, w4_body, 0)

        def wc_body(wc, carry):
            cs = pl.ds(112, 16)
            for r in range(4):
                ob[r, wc, cs] = ib[r, wc, cs]
            return carry
        lax.fori_loop(0, _CW, wc_body, 0)

    ins = (in0, in1)
    outs = (out0, out1)
    sins = (si0, si1)
    souts = (so0, so1)

    for b in range(2):
        pltpu.make_async_copy(chunk_slice(x_hbm, b), ins[b], sins[b]).start()

    def step(p, carry):
        for b in range(2):
            k = p * 2 + b
            ib, ob, si, so = ins[b], outs[b], sins[b], souts[b]
            pltpu.make_async_copy(chunk_slice(x_hbm, k), ib, si).wait()
            pl.when(p > 0)(
                lambda ob=ob, so=so, k=k:
                    pltpu.make_async_copy(ob, chunk_slice(y_hbm, k - 2), so).wait())
            compute(ib, ob)
            pltpu.make_async_copy(ob, chunk_slice(y_hbm, k), so).start()
            pl.when(p < _PAIRS - 1)(
                lambda ib=ib, si=si, k=k:
                    pltpu.make_async_copy(chunk_slice(x_hbm, k + 2), ib, si).start())
        return carry

    lax.fori_loop(0, _PAIRS, step, 0)

    for b in range(2):
        k = _CHUNKS - 2 + b
        pltpu.make_async_copy(outs[b], chunk_slice(y_hbm, k), souts[b]).wait()


def kernel(activation):
    xt = jnp.transpose(activation, (0, 2, 3, 1))
    yt = _block_relu_nhwc(xt)
    return jnp.transpose(yt, (0, 3, 1, 2))
